# Initial kernel scaffold; baseline (speedup 1.0000x reference)
import functools

import jax
import jax.numpy as jnp
from jax.experimental import pallas as pl
from jax.experimental.pallas import tpu as pltpu

_HIDDEN = 768
_E = 8
_NT = 32768
_BT = 1024
_GRID = _NT // _BT


def _router_body(x_ref, wt_ref, idx_ref, w_ref, cnt_ref, avg_ref, ent_ref,
                 gini_ref):
    i = pl.program_id(0)
    x = x_ref[...]
    wt = wt_ref[...]

    logits = jax.lax.dot_general(
        x, wt, (((1,), (0,)), ((), ())),
        preferred_element_type=jnp.float32)

    m = jnp.max(logits, axis=-1, keepdims=True)
    e = jnp.exp(logits - m)
    z = jnp.sum(e, axis=-1, keepdims=True)
    probs = e / z

    p1 = jnp.max(probs, axis=-1, keepdims=True)
    e1 = jnp.argmax(probs, axis=-1).astype(jnp.int32)[:, None]
    iota = jax.lax.broadcasted_iota(jnp.int32, (_BT, _E), 1)
    masked = jnp.where(iota == e1, -jnp.inf, probs)
    p2 = jnp.max(masked, axis=-1, keepdims=True)
    e2 = jnp.argmax(masked, axis=-1).astype(jnp.int32)[:, None]

    wsum = p1 + p2
    idx_ref[...] = jnp.concatenate([e1, e2], axis=1)
    w_ref[...] = jnp.concatenate([p1 / wsum, p2 / wsum], axis=1)

    onehot = (iota == e1).astype(jnp.float32) + (iota == e2).astype(jnp.float32)
    cnt_part = jnp.sum(onehot, axis=0, keepdims=True)
    p_part = jnp.sum(probs, axis=0, keepdims=True)
    ent_part = -jnp.sum(probs * jnp.log(probs + 1e-10))

    @pl.when(i == 0)
    def _init():
        cnt_ref[...] = cnt_part
        avg_ref[...] = p_part
        ent_ref[0, 0] = ent_part

    @pl.when(i > 0)
    def _acc():
        cnt_ref[...] += cnt_part
        avg_ref[...] += p_part
        ent_ref[0, 0] += ent_part

    @pl.when(i == _GRID - 1)
    def _final():
        avg_ref[...] = avg_ref[...] / _NT
        ent_ref[0, 0] = ent_ref[0, 0] / _NT
        c = cnt_ref[...]
        pair = jnp.sum(jnp.abs(c[:, :, None] - c[:, None, :]))
        gini_ref[0, 0] = 0.5 * pair / (_E * jnp.sum(c) + 1e-10)


@jax.jit
def _router(x, wt):
    grid = (_GRID,)
    out_shapes = (
        jax.ShapeDtypeStruct((_NT, 2), jnp.int32),
        jax.ShapeDtypeStruct((_NT, 2), jnp.float32),
        jax.ShapeDtypeStruct((1, _E), jnp.float32),
        jax.ShapeDtypeStruct((1, _E), jnp.float32),
        jax.ShapeDtypeStruct((1, 1), jnp.float32),
        jax.ShapeDtypeStruct((1, 1), jnp.float32),
    )
    out_specs = (
        pl.BlockSpec((_BT, 2), lambda i: (i, 0)),
        pl.BlockSpec((_BT, 2), lambda i: (i, 0)),
        pl.BlockSpec((1, _E), lambda i: (0, 0)),
        pl.BlockSpec((1, _E), lambda i: (0, 0)),
        pl.BlockSpec((1, 1), lambda i: (0, 0)),
        pl.BlockSpec((1, 1), lambda i: (0, 0)),
    )
    in_specs = [
        pl.BlockSpec((_BT, _HIDDEN), lambda i: (i, 0)),
        pl.BlockSpec((_HIDDEN, _E), lambda i: (0, 0)),
    ]
    return pl.pallas_call(
        _router_body,
        grid=grid,
        in_specs=in_specs,
        out_specs=out_specs,
        out_shape=out_shapes,
    )(x, wt)


def kernel(x, W):
    idx, wts, cnt, avg, ent, gini = _router(x, W.T)
    return (idx, wts, cnt.reshape(_E), avg.reshape(_E),
            ent.reshape(()), gini.reshape(()))


# speedup vs baseline: 1.3000x; 1.3000x over previous
import functools

import jax
import jax.numpy as jnp
from jax.experimental import pallas as pl
from jax.experimental.pallas import tpu as pltpu

_HIDDEN = 768
_E = 8
_NT = 32768
_BT = 1024
_GRID = _NT // _BT


def _router_body(x_ref, wt_ref, idx_ref, w_ref, cnt_ref, avg_ref, ent_ref,
                 gini_ref):
    i = pl.program_id(0)
    x = x_ref[...]
    wt = wt_ref[...]

    logits = jax.lax.dot_general(
        x, wt, (((1,), (0,)), ((), ())),
        preferred_element_type=jnp.float32)

    m = jnp.max(logits, axis=-1, keepdims=True)
    e = jnp.exp(logits - m)
    z = jnp.sum(e, axis=-1, keepdims=True)
    probs = e / z

    p1 = jnp.max(probs, axis=-1, keepdims=True)
    e1 = jnp.argmax(probs, axis=-1).astype(jnp.int32)[:, None]
    iota = jax.lax.broadcasted_iota(jnp.int32, (_BT, _E), 1)
    masked = jnp.where(iota == e1, -jnp.inf, probs)
    p2 = jnp.max(masked, axis=-1, keepdims=True)
    e2 = jnp.argmax(masked, axis=-1).astype(jnp.int32)[:, None]

    wsum = p1 + p2
    idx_ref[...] = jnp.concatenate([e1, e2], axis=1)
    w_ref[...] = jnp.concatenate([p1 / wsum, p2 / wsum], axis=1)

    onehot = (iota == e1).astype(jnp.float32) + (iota == e2).astype(jnp.float32)
    cnt_part = jnp.sum(onehot, axis=0, keepdims=True)
    p_part = jnp.sum(probs, axis=0, keepdims=True)
    ent_part = -jnp.sum(probs * jnp.log(probs + 1e-10)).reshape(1, 1)

    @pl.when(i == 0)
    def _init():
        cnt_ref[...] = cnt_part
        avg_ref[...] = p_part
        ent_ref[...] = ent_part

    @pl.when(i > 0)
    def _acc():
        cnt_ref[...] += cnt_part
        avg_ref[...] += p_part
        ent_ref[...] += ent_part

    @pl.when(i == _GRID - 1)
    def _final():
        avg_ref[...] = avg_ref[...] / _NT
        ent_ref[...] = ent_ref[...] / _NT
        c = cnt_ref[...]
        pair = jnp.sum(jnp.abs(c[:, :, None] - c[:, None, :])).reshape(1, 1)
        gini_ref[...] = 0.5 * pair / (_E * jnp.sum(c) + 1e-10)


@jax.jit
def _router(x, wt):
    grid = (_GRID,)
    out_shapes = (
        jax.ShapeDtypeStruct((_NT, 2), jnp.int32),
        jax.ShapeDtypeStruct((_NT, 2), jnp.float32),
        jax.ShapeDtypeStruct((1, _E), jnp.float32),
        jax.ShapeDtypeStruct((1, _E), jnp.float32),
        jax.ShapeDtypeStruct((1, 1), jnp.float32),
        jax.ShapeDtypeStruct((1, 1), jnp.float32),
    )
    out_specs = (
        pl.BlockSpec((_BT, 2), lambda i: (i, 0)),
        pl.BlockSpec((_BT, 2), lambda i: (i, 0)),
        pl.BlockSpec((1, _E), lambda i: (0, 0)),
        pl.BlockSpec((1, _E), lambda i: (0, 0)),
        pl.BlockSpec((1, 1), lambda i: (0, 0)),
        pl.BlockSpec((1, 1), lambda i: (0, 0)),
    )
    in_specs = [
        pl.BlockSpec((_BT, _HIDDEN), lambda i: (i, 0)),
        pl.BlockSpec((_HIDDEN, _E), lambda i: (0, 0)),
    ]
    return pl.pallas_call(
        _router_body,
        grid=grid,
        in_specs=in_specs,
        out_specs=out_specs,
        out_shape=out_shapes,
    )(x, wt)


def kernel(x, W):
    idx, wts, cnt, avg, ent, gini = _router(x, W.T)
    return (idx, wts, cnt.reshape(_E), avg.reshape(_E),
            ent.reshape(()), gini.reshape(()))


# speedup vs baseline: 1.4933x; 1.1487x over previous
import functools

import jax
import jax.numpy as jnp
from jax.experimental import pallas as pl
from jax.experimental.pallas import tpu as pltpu

_HIDDEN = 768
_E = 8
_NT = 32768
_BT = 2048
_GRID = _NT // _BT


def _router_body(x_ref, wt_ref, idx_ref, w_ref, cnt_ref, avg_ref, ent_ref,
                 gini_ref):
    i = pl.program_id(0)
    x = x_ref[...]
    wt = wt_ref[...]

    logits = jax.lax.dot_general(
        x, wt, (((1,), (0,)), ((), ())),
        preferred_element_type=jnp.float32)

    m = jnp.max(logits, axis=-1, keepdims=True)
    e = jnp.exp(logits - m)
    z = jnp.sum(e, axis=-1, keepdims=True)
    probs = e / z

    p1 = jnp.max(probs, axis=-1, keepdims=True)
    e1 = jnp.argmax(probs, axis=-1).astype(jnp.int32)[:, None]
    iota = jax.lax.broadcasted_iota(jnp.int32, (_BT, _E), 1)
    masked = jnp.where(iota == e1, -jnp.inf, probs)
    p2 = jnp.max(masked, axis=-1, keepdims=True)
    e2 = jnp.argmax(masked, axis=-1).astype(jnp.int32)[:, None]

    wsum = p1 + p2
    idx_ref[...] = jnp.concatenate([e1, e2], axis=1)
    w_ref[...] = jnp.concatenate([p1 / wsum, p2 / wsum], axis=1)

    onehot = (iota == e1).astype(jnp.float32) + (iota == e2).astype(jnp.float32)
    cnt_part = jnp.sum(onehot, axis=0, keepdims=True)
    p_part = jnp.sum(probs, axis=0, keepdims=True)
    ent_part = -jnp.sum(probs * jnp.log(probs + 1e-10)).reshape(1, 1)

    @pl.when(i == 0)
    def _init():
        cnt_ref[...] = cnt_part
        avg_ref[...] = p_part
        ent_ref[...] = ent_part

    @pl.when(i > 0)
    def _acc():
        cnt_ref[...] += cnt_part
        avg_ref[...] += p_part
        ent_ref[...] += ent_part

    @pl.when(i == _GRID - 1)
    def _final():
        avg_ref[...] = avg_ref[...] / _NT
        ent_ref[...] = ent_ref[...] / _NT
        c = cnt_ref[...]
        pair = jnp.sum(jnp.abs(c[:, :, None] - c[:, None, :])).reshape(1, 1)
        gini_ref[...] = 0.5 * pair / (_E * jnp.sum(c) + 1e-10)


@jax.jit
def _router(x, wt):
    grid = (_GRID,)
    out_shapes = (
        jax.ShapeDtypeStruct((_NT, 2), jnp.int32),
        jax.ShapeDtypeStruct((_NT, 2), jnp.float32),
        jax.ShapeDtypeStruct((1, _E), jnp.float32),
        jax.ShapeDtypeStruct((1, _E), jnp.float32),
        jax.ShapeDtypeStruct((1, 1), jnp.float32),
        jax.ShapeDtypeStruct((1, 1), jnp.float32),
    )
    out_specs = (
        pl.BlockSpec((_BT, 2), lambda i: (i, 0)),
        pl.BlockSpec((_BT, 2), lambda i: (i, 0)),
        pl.BlockSpec((1, _E), lambda i: (0, 0)),
        pl.BlockSpec((1, _E), lambda i: (0, 0)),
        pl.BlockSpec((1, 1), lambda i: (0, 0)),
        pl.BlockSpec((1, 1), lambda i: (0, 0)),
    )
    in_specs = [
        pl.BlockSpec((_BT, _HIDDEN), lambda i: (i, 0)),
        pl.BlockSpec((_HIDDEN, _E), lambda i: (0, 0)),
    ]
    return pl.pallas_call(
        _router_body,
        grid=grid,
        in_specs=in_specs,
        out_specs=out_specs,
        out_shape=out_shapes,
    )(x, wt)


def kernel(x, W):
    idx, wts, cnt, avg, ent, gini = _router(x, W.T)
    return (idx, wts, cnt.reshape(_E), avg.reshape(_E),
            ent.reshape(()), gini.reshape(()))


# speedup vs baseline: 1.5468x; 1.0359x over previous
import functools

import jax
import jax.numpy as jnp
from jax.experimental import pallas as pl
from jax.experimental.pallas import tpu as pltpu

_HIDDEN = 768
_E = 8
_NT = 32768
_BT = 4096
_GRID = _NT // _BT


def _router_body(x_ref, wt_ref, idx_ref, w_ref, cnt_ref, avg_ref, ent_ref,
                 gini_ref):
    i = pl.program_id(0)
    x = x_ref[...]
    wt = wt_ref[...]

    logits = jax.lax.dot_general(
        x, wt, (((1,), (0,)), ((), ())),
        preferred_element_type=jnp.float32)

    m = jnp.max(logits, axis=-1, keepdims=True)
    e = jnp.exp(logits - m)
    z = jnp.sum(e, axis=-1, keepdims=True)
    probs = e / z

    p1 = jnp.max(probs, axis=-1, keepdims=True)
    e1 = jnp.argmax(probs, axis=-1).astype(jnp.int32)[:, None]
    iota = jax.lax.broadcasted_iota(jnp.int32, (_BT, _E), 1)
    masked = jnp.where(iota == e1, -jnp.inf, probs)
    p2 = jnp.max(masked, axis=-1, keepdims=True)
    e2 = jnp.argmax(masked, axis=-1).astype(jnp.int32)[:, None]

    wsum = p1 + p2
    idx_ref[...] = jnp.concatenate([e1, e2], axis=1)
    w_ref[...] = jnp.concatenate([p1 / wsum, p2 / wsum], axis=1)

    onehot = (iota == e1).astype(jnp.float32) + (iota == e2).astype(jnp.float32)
    cnt_part = jnp.sum(onehot, axis=0, keepdims=True)
    p_part = jnp.sum(probs, axis=0, keepdims=True)
    ent_part = -jnp.sum(probs * jnp.log(probs + 1e-10)).reshape(1, 1)

    @pl.when(i == 0)
    def _init():
        cnt_ref[...] = cnt_part
        avg_ref[...] = p_part
        ent_ref[...] = ent_part

    @pl.when(i > 0)
    def _acc():
        cnt_ref[...] += cnt_part
        avg_ref[...] += p_part
        ent_ref[...] += ent_part

    @pl.when(i == _GRID - 1)
    def _final():
        avg_ref[...] = avg_ref[...] / _NT
        ent_ref[...] = ent_ref[...] / _NT
        c = cnt_ref[...]
        pair = jnp.sum(jnp.abs(c[:, :, None] - c[:, None, :])).reshape(1, 1)
        gini_ref[...] = 0.5 * pair / (_E * jnp.sum(c) + 1e-10)


@jax.jit
def _router(x, wt):
    grid = (_GRID,)
    out_shapes = (
        jax.ShapeDtypeStruct((_NT, 2), jnp.int32),
        jax.ShapeDtypeStruct((_NT, 2), jnp.float32),
        jax.ShapeDtypeStruct((1, _E), jnp.float32),
        jax.ShapeDtypeStruct((1, _E), jnp.float32),
        jax.ShapeDtypeStruct((1, 1), jnp.float32),
        jax.ShapeDtypeStruct((1, 1), jnp.float32),
    )
    out_specs = (
        pl.BlockSpec((_BT, 2), lambda i: (i, 0)),
        pl.BlockSpec((_BT, 2), lambda i: (i, 0)),
        pl.BlockSpec((1, _E), lambda i: (0, 0)),
        pl.BlockSpec((1, _E), lambda i: (0, 0)),
        pl.BlockSpec((1, 1), lambda i: (0, 0)),
        pl.BlockSpec((1, 1), lambda i: (0, 0)),
    )
    in_specs = [
        pl.BlockSpec((_BT, _HIDDEN), lambda i: (i, 0)),
        pl.BlockSpec((_HIDDEN, _E), lambda i: (0, 0)),
    ]
    return pl.pallas_call(
        _router_body,
        grid=grid,
        in_specs=in_specs,
        out_specs=out_specs,
        out_shape=out_shapes,
    )(x, wt)


def kernel(x, W):
    idx, wts, cnt, avg, ent, gini = _router(x, W.T)
    return (idx, wts, cnt.reshape(_E), avg.reshape(_E),
            ent.reshape(()), gini.reshape(()))
